# Initial kernel scaffold; baseline (speedup 1.0000x reference)
import jax
import jax.numpy as jnp
from jax.experimental import pallas as pl


def kernel(x, edge_index, gate, forward_level, params):
    raise NotImplementedError("write your pallas kernel here")


# speedup vs baseline: 46.0420x; 46.0420x over previous
import functools

import jax
import jax.numpy as jnp
from jax import lax
from jax.experimental import pallas as pl
from jax.experimental.pallas import tpu as pltpu
from jax.experimental.pallas import tpu_sc as plsc

H = 64
N = 10000
E = 320000
NC = 2
NS = 16
NW = NC * NS
ET = E // NW
ETP = 10112
CH = 128
TRASH = N
NR = 10112
RPT = NR // NS
GW = 128
ES_LEN = NW * ETP + CH

_MESH = plsc.VectorSubcoreMesh(core_axis_name="c", subcore_axis_name="s")


def _sort_body(ei, gate_h, fl_h, es_src, es_dst, offs,
               gate_v, fl_v, src_v, dst_v, osrc_v, odst_v, offs_v, zpad_v):
    cid = lax.axis_index("c")
    sid = lax.axis_index("s")
    w = sid * NC + cid
    pltpu.sync_copy(gate_h, gate_v)
    pltpu.sync_copy(fl_h, fl_v)
    pltpu.sync_copy(ei.at[pl.ds(w * ET, ET)], src_v)
    pltpu.sync_copy(ei.at[pl.ds(E + w * ET, ET)], dst_v)
    lanes = lax.iota(jnp.int32, 16)

    def key_at(j):
        d = dst_v[pl.ds(j * 16, 16)]
        g = plsc.load_gather(gate_v, [d])
        f = plsc.load_gather(fl_v, [d])
        return jnp.where((g >= 1) & (g <= 3), f * 3 + (g - 1), 12)

    def h_body(j, accs):
        key = key_at(j)
        return tuple(accs[t] + (key == t).astype(jnp.int32) for t in range(12))

    accs = lax.fori_loop(0, ET // 16, h_body,
                         tuple([jnp.zeros((16,), jnp.int32)] * 12))
    cnts = [jnp.sum(a) for a in accs]

    starts = [jnp.int32(0)]
    for t in range(12):
        starts.append(((starts[t] + cnts[t]) + 7) & (-8))
    ov = jnp.zeros((16,), jnp.int32)
    for t in range(13):
        ov = jnp.where(lanes == t, w * ETP + starts[t], ov)
    for t in range(13, 16):
        ov = jnp.where(lanes == t, w * ETP + starts[12], ov)
    offs_v[...] = ov
    pltpu.sync_copy(offs_v, offs.at[w])

    zeros16 = jnp.zeros((16,), jnp.int32)
    trash16 = jnp.full((16,), TRASH, jnp.int32)

    def i_body(j, _):
        osrc_v[pl.ds(j * 16, 16)] = zeros16
        odst_v[pl.ds(j * 16, 16)] = trash16
        return 0

    lax.fori_loop(0, ETP // 16, i_body, 0)

    def s_body(j, cursors):
        key = key_at(j)
        src = src_v[pl.ds(j * 16, 16)]
        dst = dst_v[pl.ds(j * 16, 16)]
        new = []
        for t in range(12):
            m = key == t
            mi = m.astype(jnp.int32)
            pos = cursors[t] + jnp.cumsum(mi) - 1
            plsc.store_scatter(osrc_v, [pos], src, mask=m)
            plsc.store_scatter(odst_v, [pos], dst, mask=m)
            new.append(cursors[t] + jnp.sum(mi))
        return tuple(new)

    lax.fori_loop(0, ET // 16, s_body, tuple(starts[:12]))

    pltpu.sync_copy(osrc_v, es_src.at[pl.ds(w * ETP, ETP)])
    pltpu.sync_copy(odst_v, es_dst.at[pl.ds(w * ETP, ETP)])

    @pl.when(w == 0)
    def _():
        def z_body(j, _):
            zpad_v[pl.ds(j * 16, 16)] = zeros16
            return 0
        lax.fori_loop(0, CH // 16, z_body, 0)
        pltpu.sync_copy(zpad_v, es_src.at[pl.ds(NW * ETP, CH)])
        pltpu.sync_copy(zpad_v, es_dst.at[pl.ds(NW * ETP, CH)])


def _sort_edges(ei, gate, fl):
    f = pl.kernel(
        _sort_body,
        out_type=(
            jax.ShapeDtypeStruct((ES_LEN,), jnp.int32),
            jax.ShapeDtypeStruct((ES_LEN,), jnp.int32),
            jax.ShapeDtypeStruct((NW, 16), jnp.int32),
        ),
        mesh=_MESH,
        scratch_types=[
            pltpu.VMEM((N,), jnp.int32),
            pltpu.VMEM((N,), jnp.int32),
            pltpu.VMEM((ET,), jnp.int32),
            pltpu.VMEM((ET,), jnp.int32),
            pltpu.VMEM((ETP,), jnp.int32),
            pltpu.VMEM((ETP,), jnp.int32),
            pltpu.VMEM((16,), jnp.int32),
            pltpu.VMEM((CH,), jnp.int32),
        ],
        name="edge_bucket_sort",
        compiler_params=pltpu.CompilerParams(needs_layout_passes=False),
    )
    return f(ei, gate, fl)


def _scatter_body(g_rows, es_src, es_dst, offs, tvec, zrows, out,
                  idx_s, idx_d, rows_v, offs_v, tv_v, sem, msg_sh):
    cid = lax.axis_index("c")
    sid = lax.axis_index("s")
    w = sid * NC + cid
    lanes = lax.iota(jnp.int32, 16)

    pltpu.sync_copy(zrows.at[pl.ds(sid * RPT, RPT)],
                    msg_sh.at[pl.ds(sid * RPT, RPT)])
    pltpu.sync_copy(offs.at[w], offs_v)
    pltpu.sync_copy(tvec, tv_v)
    plsc.subcore_barrier()

    ov = offs_v[...]
    t_s = jnp.sum(jnp.where(lanes == 0, tv_v[...], 0))
    lo = jnp.sum(jnp.where(lanes == t_s, ov, 0))
    hi = jnp.sum(jnp.where(lanes == t_s + 1, ov, 0))

    def chunk(i, _):
        base = pl.multiple_of(lo + i * CH, 8)

        @pl.when(base < hi)
        def _():
            pltpu.sync_copy(es_src.at[pl.ds(base, CH)], idx_s)
            pltpu.sync_copy(es_dst.at[pl.ds(base, CH)], idx_d)

            def fix(k, _2):
                dd = idx_d[pl.ds(k * 16, 16)]
                pos = base + k * 16 + lanes
                idx_d[pl.ds(k * 16, 16)] = jnp.where(pos < hi, dd, TRASH)
                return 0
            lax.fori_loop(0, CH // 16, fix, 0)

            pltpu.async_copy(g_rows.at[idx_s], rows_v, sem).wait()
            pltpu.sync_copy(rows_v, msg_sh.at[idx_d], add=True)
        return 0

    lax.fori_loop(0, ETP // CH, chunk, 0)
    plsc.subcore_barrier()
    pltpu.sync_copy(msg_sh.at[pl.ds(sid * RPT, RPT)],
                    out.at[cid, pl.ds(sid * RPT, RPT)])


def _scatter_rows(g_rows, es_src, es_dst, offs, tvec, zrows):
    f = pl.kernel(
        _scatter_body,
        out_type=jax.ShapeDtypeStruct((NC, NR, GW), jnp.float32),
        mesh=_MESH,
        scratch_types=[
            pltpu.VMEM((CH,), jnp.int32),
            pltpu.VMEM((CH,), jnp.int32),
            pltpu.VMEM((CH, GW), jnp.float32),
            pltpu.VMEM((16,), jnp.int32),
            pltpu.VMEM((16,), jnp.int32),
            pltpu.SemaphoreType.DMA,
            pltpu.VMEM_SHARED((NR, GW), jnp.float32),
        ],
        name="seg_scatter_add",
        compiler_params=pltpu.CompilerParams(needs_layout_passes=False),
    )
    return f(g_rows, es_src, es_dst, offs, tvec, zrows)


def _dot(a, b):
    return jnp.dot(a, b, preferred_element_type=jnp.float32)


def _gru(msg, h, wi3, wh3, bi3, bh3):
    gir = _dot(msg, wi3[0]) + bi3[0]
    giz = _dot(msg, wi3[1]) + bi3[1]
    gin = _dot(msg, wi3[2]) + bi3[2]
    ghr = _dot(h, wh3[0]) + bh3[0]
    ghz = _dot(h, wh3[1]) + bh3[1]
    ghn = _dot(h, wh3[2]) + bh3[2]
    r = jax.nn.sigmoid(gir + ghr)
    z = jax.nn.sigmoid(giz + ghz)
    n = jnp.tanh(gin + r * ghn)
    return (1.0 - z) * n + z * h


def _prologue_body(one_r, whs, bhs, whf, bhf, whq, bhq, s123, sb,
                   hs_o, hf_o, hq_o, gs_o, hs0_o, hf0_o):
    one = one_r[...]
    hs0 = one * whs[...] + bhs[...]
    hf0 = one * whf[...] + bhf[...]
    hq0 = one * whq[...] + bhq[...]
    g = jax.nn.relu(_dot(hs0, s123[0]) + sb[0])
    g = jax.nn.relu(_dot(g, s123[1]) + sb[1])
    g = _dot(g, s123[2]) + sb[2]
    hs_o[...] = jnp.broadcast_to(hs0, (N, H))
    hf_o[...] = jnp.broadcast_to(hf0, (N, H))
    hq_o[...] = jnp.broadcast_to(hq0, (N, H))
    gs_o[...] = jnp.broadcast_to(_widen(g), (N, GW))
    hs0_o[...] = hs0
    hf0_o[...] = hf0


def _widen(g):
    lane = lax.broadcasted_iota(jnp.int32, (1, H), 1)
    onecol = jnp.where(lane == 0, 1.0, 0.0).astype(jnp.float32)
    return jnp.concatenate(
        [g, jnp.broadcast_to(onecol, (g.shape[0], H))], axis=1)


def _iter_body(has_prev, has_next, *refs):
    it = iter(refs)
    nxt = lambda: next(it)
    hs_r, hf_r, hq_r, gate_r, fl_r, lvl_r, gid_r = (nxt() for _ in range(7))
    hs0_r, hf0_r = nxt(), nxt()
    msgs0_r, msgs1_r = nxt(), nxt()
    swi, swh, sbi, sbh = (nxt() for _ in range(4))
    fwi, fwh, fbi, fbh = (nxt() for _ in range(4))
    f1s, f23, fb = (nxt() for _ in range(3))
    q1s, q2, q3, qb = (nxt() for _ in range(4))
    if has_prev:
        plvl_r, pgid_r, msgq0_r, msgq1_r = (nxt() for _ in range(4))
        pwi, pwh, pbi, pbh = (nxt() for _ in range(4))
    if has_next:
        n123, nb = nxt(), nxt()
    hs_o, hf_o, hq_o, gq_o = (nxt() for _ in range(4))
    if has_next:
        gs_o = nxt()

    gate = gate_r[...]
    fl = fl_r[...]
    hq = hq_r[...]
    if has_prev:
        pm = (gate == pgid_r[...]) & (fl == plvl_r[...])
        msgq = msgq0_r[:, 0:H] + msgq1_r[:, 0:H]
        hq = jnp.where(pm, _gru(msgq, hq, pwi, pwh, pbi, pbh), hq)
    nm = (gate == gid_r[...]) & (fl == lvl_r[...])
    msgs = msgs0_r[:, 0:H] + msgs1_r[:, 0:H]
    hs = hs_r[...]
    hs2 = jnp.where(nm, _gru(msgs, hs, swi, swh, sbi, sbh), hs)
    deg = msgs0_r[:, H:H + 1] + msgs1_r[:, H:H + 1]
    vf = jax.nn.relu(_dot(hs0_r[...], f1s[0]) + _dot(hf0_r[...], f1s[1]) + fb[0])
    vf = jax.nn.relu(_dot(vf, f23[0]) + fb[1])
    vf = _dot(vf, f23[1]) + fb[2]
    msgf = deg * vf
    hf = hf_r[...]
    hf2 = jnp.where(nm, _gru(msgf, hf, fwi, fwh, fbi, fbh), hf)
    u = _dot(hs2, q1s[0]) + _dot(hf2, q1s[1]) + _dot(hq, q1s[2]) + qb[0]
    u = jax.nn.relu(u)
    u = jax.nn.relu(_dot(u, q2[...]) + qb[1])
    gq_o[...] = _widen(_dot(u, q3[...]) + qb[2])
    if has_next:
        v = jax.nn.relu(_dot(hs2, n123[0]) + nb[0])
        v = jax.nn.relu(_dot(v, n123[1]) + nb[1])
        gs_o[...] = _widen(_dot(v, n123[2]) + nb[2])
    hs_o[...] = hs2
    hf_o[...] = hf2
    hq_o[...] = hq


def _epilogue_body(hf_r, hq_r, gate_r, fl_r, plvl_r, pgid_r,
                   msgq0_r, msgq1_r, pwi, pwh, pbi, pbh,
                   a01, a02, b01, b02, a11, a12, b11, b12,
                   at1, at2, bt1, bt2, a3s, b4, out_r):
    gate = gate_r[...]
    fl = fl_r[...]
    pm = (gate == pgid_r[...]) & (fl == plvl_r[...])
    msgq = msgq0_r[:, 0:H] + msgq1_r[:, 0:H]
    hq = hq_r[...]
    hq = jnp.where(pm, _gru(msgq, hq, pwi, pwh, pbi, pbh), hq)
    hf = hf_r[...]
    h0 = jax.nn.relu(_dot(hf, a01[...]) + b01[...])
    h0 = jax.nn.relu(_dot(h0, a02[...]) + b02[...])
    h1 = jax.nn.relu(_dot(hf, a11[...]) + b11[...])
    h1 = jax.nn.relu(_dot(h1, a12[...]) + b12[...])
    ht = jax.nn.relu(_dot(hq, at1[...]) + bt1[...])
    ht = jax.nn.relu(_dot(ht, at2[...]) + bt2[...])
    out_r[...] = (_dot(h0, a3s[0]) + _dot(h1, a3s[1]) + _dot(ht, a3s[2])
                  + b4[...])


def _tc_call(body, out_shapes, name):
    return pl.pallas_call(body, out_shape=out_shapes, name=name)


BN = 2000


def _row_spec(shape):
    if shape[0] in (N, NR):
        nd = len(shape)
        return pl.BlockSpec((BN,) + tuple(shape[1:]),
                            lambda i, _nd=nd: (i,) + (0,) * (_nd - 1))
    nd = len(shape)
    return pl.BlockSpec(tuple(shape), lambda i, _nd=nd: (0,) * _nd)


def _tc_call_grid(body, out_shapes, name, args):
    single = not isinstance(out_shapes, (list, tuple))
    outs = [out_shapes] if single else list(out_shapes)
    f = pl.pallas_call(
        body,
        out_shape=out_shapes,
        grid=(N // BN,),
        in_specs=[_row_spec(a.shape) for a in args],
        out_specs=(_row_spec(outs[0].shape) if single
                   else [_row_spec(o.shape) for o in outs]),
        name=name,
    )
    return f(*args)


def _gru_w(p):
    wi3 = jnp.stack(jnp.split(p['Wi'], 3, axis=1))
    wh3 = jnp.stack(jnp.split(p['Wh'], 3, axis=1))
    bi3 = p['bi'].reshape(3, 1, H)
    bh3 = p['bh'].reshape(3, 1, H)
    return [wi3, wh3, bi3, bh3]


def _mlp3_w(p):
    (w1, b1), (w2, b2), (w3, b3) = p
    return jnp.stack([w1, w2, w3]), jnp.stack(
        [b1.reshape(1, H), b2.reshape(1, H), b3.reshape(1, H)])


_NAMES = ['and', 'not', 'ff']


def kernel(x, edge_index, gate, forward_level, params):
    ei = edge_index.astype(jnp.int32).reshape(2 * E)
    g32 = gate.astype(jnp.int32)
    l32 = forward_level.astype(jnp.int32)
    es_src, es_dst, offs = _sort_edges(ei, g32, l32)
    gate2 = g32.reshape(N, 1)
    fl2 = l32.reshape(N, 1)
    zrows = jnp.zeros((NR, GW), jnp.float32)

    def sc11(v):
        return jnp.full((1, 1), v, jnp.int32)

    one = jnp.ones((1, 1), jnp.float32)
    s123_0, sb_0 = _mlp3_w(params['and_strc_aggr'])
    hs, hf, hq, gs, hs0r, hf0r = _tc_call(
        _prologue_body,
        [jax.ShapeDtypeStruct((N, H), jnp.float32)] * 3
        + [jax.ShapeDtypeStruct((N, GW), jnp.float32)]
        + [jax.ShapeDtypeStruct((1, H), jnp.float32)] * 2,
        "mlpgate_prologue")(
            one,
            params['hs_emd'][0][0], params['hs_emd'][0][1].reshape(1, H),
            params['hf_emd'][0][0], params['hf_emd'][0][1].reshape(1, H),
            params['hseq_emd'][0][0], params['hseq_emd'][0][1].reshape(1, H),
            s123_0, sb_0)

    msgq01 = None
    prev = None
    nh = jax.ShapeDtypeStruct((N, H), jnp.float32)
    for t in range(12):
        lvl, gi = t // 3, (t % 3) + 1
        name = _NAMES[t % 3]
        tvec = jnp.full((16,), t, jnp.int32)
        msgs = _scatter_rows(gs, es_src, es_dst, offs, tvec, zrows)
        args = [hs, hf, hq, gate2, fl2, sc11(lvl), sc11(gi), hs0r, hf0r,
                msgs[0], msgs[1]]
        args += _gru_w(params[name + '_strc_gru'])
        args += _gru_w(params[name + '_func_gru'])
        (fw1, fb1), (fw2, fb2), (fw3, fb3) = params[name + '_func_aggr']
        args += [jnp.stack(jnp.split(fw1, 2, axis=0)),
                 jnp.stack([fw2, fw3]),
                 jnp.stack([fb1.reshape(1, H), fb2.reshape(1, H),
                            fb3.reshape(1, H)])]
        (qw1, qb1), (qw2, qb2), (qw3, qb3) = params[name + '_seq_aggr']
        args += [jnp.stack(jnp.split(qw1, 3, axis=0)), qw2, qw3,
                 jnp.stack([qb1.reshape(1, H), qb2.reshape(1, H),
                            qb3.reshape(1, H)])]
        has_prev = prev is not None
        if has_prev:
            plvl, pgi, pname = prev
            args += [sc11(plvl), sc11(pgi), msgq01[0], msgq01[1]]
            args += _gru_w(params[pname + '_seq_gru'])
        has_next = t < 11
        if has_next:
            nname = _NAMES[(t + 1) % 3]
            n123, nb = _mlp3_w(params[nname + '_strc_aggr'])
            args += [n123, nb]
        ng = jax.ShapeDtypeStruct((N, GW), jnp.float32)
        outs = [nh, nh, nh, ng] + ([ng] if has_next else [])
        res = _tc_call_grid(
            functools.partial(_iter_body, has_prev, has_next), outs,
            f"mlpgate_iter_{int(has_prev)}{int(has_next)}", args)
        if has_next:
            hs, hf, hq, gq, gs = res
        else:
            hs, hf, hq, gq = res
        msgq01 = _scatter_rows(gq, es_src, es_dst, offs, tvec, zrows)
        prev = (lvl, gi, name)

    (a01, b01), (a02, b02), (a03, b03) = params['readout_prob0']
    (a11, b11), (a12, b12), (a13, b13) = params['readout_prob1']
    (at1, bt1), (at2, bt2), (at3, bt3) = params['readout_trans']
    a3s = jnp.stack([
        jnp.pad(a03, ((0, 0), (0, 3))),
        jnp.pad(a13, ((0, 0), (1, 2))),
        jnp.pad(at3, ((0, 0), (2, 0))),
    ])
    b4 = jnp.concatenate([b03, b13, bt3]).reshape(1, 4)
    plvl, pgi, pname = prev
    eargs = [hf, hq, gate2, fl2, sc11(plvl), sc11(pgi),
             msgq01[0], msgq01[1], *_gru_w(params[pname + '_seq_gru']),
             a01, a02, b01.reshape(1, H), b02.reshape(1, H),
             a11, a12, b11.reshape(1, H), b12.reshape(1, H),
             at1, at2, bt1.reshape(1, H), bt2.reshape(1, H),
             a3s, b4]
    out = _tc_call_grid(_epilogue_body,
                        jax.ShapeDtypeStruct((N, 4), jnp.float32),
                        "mlpgate_epilogue", eargs)
    return out


# speedup vs baseline: 63.8664x; 1.3871x over previous
import functools

import jax
import jax.numpy as jnp
from jax import lax
from jax.experimental import pallas as pl
from jax.experimental.pallas import tpu as pltpu
from jax.experimental.pallas import tpu_sc as plsc

H = 64
N = 10000
E = 320000
NC = 2
NS = 16
NW = NC * NS
ET = E // NW
ETP = 10112
CH = 128
TRASH = N
NR = 10112
RPT = NR // NS
GW = 128
ES_LEN = NW * ETP + CH

_MESH = plsc.VectorSubcoreMesh(core_axis_name="c", subcore_axis_name="s")


def _sort_body(ei, gate_h, fl_h, es_src, es_dst, offs,
               gate_v, fl_v, src_v, dst_v, osrc_v, odst_v, offs_v, zpad_v):
    cid = lax.axis_index("c")
    sid = lax.axis_index("s")
    w = sid * NC + cid
    pltpu.sync_copy(gate_h, gate_v)
    pltpu.sync_copy(fl_h, fl_v)
    pltpu.sync_copy(ei.at[pl.ds(w * ET, ET)], src_v)
    pltpu.sync_copy(ei.at[pl.ds(E + w * ET, ET)], dst_v)
    lanes = lax.iota(jnp.int32, 16)

    def key_at(j):
        d = dst_v[pl.ds(j * 16, 16)]
        g = plsc.load_gather(gate_v, [d])
        f = plsc.load_gather(fl_v, [d])
        return jnp.where((g >= 1) & (g <= 3), f * 3 + (g - 1), 12)

    def h_body(j, accs):
        key = key_at(j)
        return tuple(accs[t] + (key == t).astype(jnp.int32) for t in range(12))

    accs = lax.fori_loop(0, ET // 16, h_body,
                         tuple([jnp.zeros((16,), jnp.int32)] * 12))
    cnts = [jnp.sum(a) for a in accs]

    starts = [jnp.int32(0)]
    for t in range(12):
        starts.append(((starts[t] + cnts[t]) + 7) & (-8))
    ov = jnp.zeros((16,), jnp.int32)
    for t in range(13):
        ov = jnp.where(lanes == t, w * ETP + starts[t], ov)
    for t in range(13, 16):
        ov = jnp.where(lanes == t, w * ETP + starts[12], ov)
    offs_v[...] = ov
    pltpu.sync_copy(offs_v, offs.at[w])

    zeros16 = jnp.zeros((16,), jnp.int32)
    trash16 = jnp.full((16,), TRASH, jnp.int32)

    def i_body(j, _):
        osrc_v[pl.ds(j * 16, 16)] = zeros16
        odst_v[pl.ds(j * 16, 16)] = trash16
        return 0

    lax.fori_loop(0, ETP // 16, i_body, 0)

    def s_body(j, cursors):
        key = key_at(j)
        src = src_v[pl.ds(j * 16, 16)]
        dst = dst_v[pl.ds(j * 16, 16)]
        new = []
        for t in range(12):
            m = key == t
            mi = m.astype(jnp.int32)
            pos = cursors[t] + jnp.cumsum(mi) - 1
            plsc.store_scatter(osrc_v, [pos], src, mask=m)
            plsc.store_scatter(odst_v, [pos], dst, mask=m)
            new.append(cursors[t] + jnp.sum(mi))
        return tuple(new)

    lax.fori_loop(0, ET // 16, s_body, tuple(starts[:12]))

    pltpu.sync_copy(osrc_v, es_src.at[pl.ds(w * ETP, ETP)])
    pltpu.sync_copy(odst_v, es_dst.at[pl.ds(w * ETP, ETP)])

    @pl.when(w == 0)
    def _():
        def z_body(j, _):
            zpad_v[pl.ds(j * 16, 16)] = zeros16
            return 0
        lax.fori_loop(0, CH // 16, z_body, 0)
        pltpu.sync_copy(zpad_v, es_src.at[pl.ds(NW * ETP, CH)])
        pltpu.sync_copy(zpad_v, es_dst.at[pl.ds(NW * ETP, CH)])


def _sort_edges(ei, gate, fl):
    f = pl.kernel(
        _sort_body,
        out_type=(
            jax.ShapeDtypeStruct((ES_LEN,), jnp.int32),
            jax.ShapeDtypeStruct((ES_LEN,), jnp.int32),
            jax.ShapeDtypeStruct((NW, 16), jnp.int32),
        ),
        mesh=_MESH,
        scratch_types=[
            pltpu.VMEM((N,), jnp.int32),
            pltpu.VMEM((N,), jnp.int32),
            pltpu.VMEM((ET,), jnp.int32),
            pltpu.VMEM((ET,), jnp.int32),
            pltpu.VMEM((ETP,), jnp.int32),
            pltpu.VMEM((ETP,), jnp.int32),
            pltpu.VMEM((16,), jnp.int32),
            pltpu.VMEM((CH,), jnp.int32),
        ],
        name="edge_bucket_sort",
        compiler_params=pltpu.CompilerParams(needs_layout_passes=False),
    )
    return f(ei, gate, fl)


MAXO = (ETP // CH + 1) // 2


def _scatter2_body(do_strc, do_seq, gs_rows, gq_rows, es_src, es_dst, offs,
                   tvec, zrows, out_s, out_q,
                   idx_s0, idx_d0, rows0, idx_s1, idx_d1, rows1,
                   offs_v, tv_v, sem0, sem1, msg_sh):
    cid = lax.axis_index("c")
    sid = lax.axis_index("s")
    lanes = lax.iota(jnp.int32, 16)
    pltpu.sync_copy(tvec, tv_v)
    t_s = jnp.sum(jnp.where(lanes == 0, tv_v[...], 0))
    bufs = ((idx_s0, idx_d0, rows0, sem0), (idx_s1, idx_d1, rows1, sem1))

    def run_side(g_ref, out_ref, tt):
        pltpu.sync_copy(zrows.at[pl.ds(sid * RPT, RPT)],
                        msg_sh.at[pl.ds(sid * RPT, RPT)])
        plsc.subcore_barrier()
        for rep in range(2):
            w = sid + NS * rep
            pltpu.sync_copy(offs.at[w], offs_v)
            ov = offs_v[...]
            lo = jnp.sum(jnp.where(lanes == tt, ov, 0))
            hi = jnp.sum(jnp.where(lanes == tt + 1, ov, 0))
            nch = (hi - lo + CH - 1) // CH

            def stage(k, b):
                isb, idb, rwb, smb = bufs[b]
                base = pl.multiple_of(lo + k * CH, 8)
                pltpu.sync_copy(es_src.at[pl.ds(base, CH)], isb)
                pltpu.sync_copy(es_dst.at[pl.ds(base, CH)], idb)

                def fix(j, _):
                    dd = idb[pl.ds(j * 16, 16)]
                    pos = base + j * 16 + lanes
                    idb[pl.ds(j * 16, 16)] = jnp.where(pos < hi, dd, TRASH)
                    return 0
                lax.fori_loop(0, CH // 16, fix, 0)
                pltpu.make_async_copy(g_ref.at[isb], rwb, smb).start()

            for b in range(2):
                @pl.when(b < nch)
                def _(b=b):
                    stage(jnp.int32(b), b)

            def outer(io, _):
                for b in range(2):
                    i = io * 2 + b
                    isb, idb, rwb, smb = bufs[b]

                    @pl.when(i < nch)
                    def _(i=i, b=b, isb=isb, idb=idb, rwb=rwb, smb=smb):
                        pltpu.make_async_copy(g_ref.at[isb], rwb, smb).wait()
                        pltpu.sync_copy(rwb, msg_sh.at[idb], add=True)

                        @pl.when(i + 2 < nch)
                        def _():
                            stage(i + 2, b)
                return 0

            lax.fori_loop(0, MAXO, outer, 0)
        plsc.subcore_barrier()
        pltpu.sync_copy(msg_sh.at[pl.ds(sid * RPT, RPT)],
                        out_ref.at[pl.ds(sid * RPT, RPT)])

    if do_strc:
        @pl.when(cid == 0)
        def _():
            run_side(gs_rows, out_s, t_s + 1)
    if do_seq:
        @pl.when(cid == 1)
        def _():
            run_side(gq_rows, out_q, t_s)


def _scatter_pair(t, gs_rows, gq_rows, es_src, es_dst, offs, zrows,
                  do_strc=True, do_seq=True):
    tvec = jnp.full((16,), t, jnp.int32)
    f = pl.kernel(
        functools.partial(_scatter2_body, do_strc, do_seq),
        out_type=(jax.ShapeDtypeStruct((NR, GW), jnp.float32),
                  jax.ShapeDtypeStruct((NR, GW), jnp.float32)),
        mesh=_MESH,
        scratch_types=[
            pltpu.VMEM((CH,), jnp.int32),
            pltpu.VMEM((CH,), jnp.int32),
            pltpu.VMEM((CH, GW), jnp.float32),
            pltpu.VMEM((CH,), jnp.int32),
            pltpu.VMEM((CH,), jnp.int32),
            pltpu.VMEM((CH, GW), jnp.float32),
            pltpu.VMEM((16,), jnp.int32),
            pltpu.VMEM((16,), jnp.int32),
            pltpu.SemaphoreType.DMA,
            pltpu.SemaphoreType.DMA,
            pltpu.VMEM_SHARED((NR, GW), jnp.float32),
        ],
        name="seg_scatter_pair",
        compiler_params=pltpu.CompilerParams(needs_layout_passes=False),
    )
    return f(gs_rows, gq_rows, es_src, es_dst, offs, tvec, zrows)


def _dot(a, b):
    return jnp.dot(a, b, preferred_element_type=jnp.float32)


def _gru(msg, h, wi3, wh3, bi3, bh3):
    gir = _dot(msg, wi3[0]) + bi3[0]
    giz = _dot(msg, wi3[1]) + bi3[1]
    gin = _dot(msg, wi3[2]) + bi3[2]
    ghr = _dot(h, wh3[0]) + bh3[0]
    ghz = _dot(h, wh3[1]) + bh3[1]
    ghn = _dot(h, wh3[2]) + bh3[2]
    r = jax.nn.sigmoid(gir + ghr)
    z = jax.nn.sigmoid(giz + ghz)
    n = jnp.tanh(gin + r * ghn)
    return (1.0 - z) * n + z * h


def _prologue_body(one_r, whs, bhs, whf, bhf, whq, bhq, s123, sb,
                   hs_o, hf_o, hq_o, gs_o, hs0_o, hf0_o):
    one = one_r[...]
    hs0 = one * whs[...] + bhs[...]
    hf0 = one * whf[...] + bhf[...]
    hq0 = one * whq[...] + bhq[...]
    g = jax.nn.relu(_dot(hs0, s123[0]) + sb[0])
    g = jax.nn.relu(_dot(g, s123[1]) + sb[1])
    g = _dot(g, s123[2]) + sb[2]
    hs_o[...] = jnp.broadcast_to(hs0, (N, H))
    hf_o[...] = jnp.broadcast_to(hf0, (N, H))
    hq_o[...] = jnp.broadcast_to(hq0, (N, H))
    gs_o[...] = jnp.broadcast_to(_widen(g), (N, GW))
    hs0_o[...] = hs0
    hf0_o[...] = hf0


def _widen(g):
    lane = lax.broadcasted_iota(jnp.int32, (1, H), 1)
    onecol = jnp.where(lane == 0, 1.0, 0.0).astype(jnp.float32)
    return jnp.concatenate(
        [g, jnp.broadcast_to(onecol, (g.shape[0], H))], axis=1)


def _iter_body(has_prev, has_next, *refs):
    it = iter(refs)
    nxt = lambda: next(it)
    hs_r, hf_r, hq_r, gate_r, fl_r, lvl_r, gid_r = (nxt() for _ in range(7))
    hs0_r, hf0_r = nxt(), nxt()
    msgs_r = nxt()
    swi, swh, sbi, sbh = (nxt() for _ in range(4))
    fwi, fwh, fbi, fbh = (nxt() for _ in range(4))
    f1s, f23, fb = (nxt() for _ in range(3))
    q1s, q2, q3, qb = (nxt() for _ in range(4))
    if has_prev:
        plvl_r, pgid_r, msgq_r = (nxt() for _ in range(3))
        pwi, pwh, pbi, pbh = (nxt() for _ in range(4))
    if has_next:
        n123, nb = nxt(), nxt()
    hs_o, hf_o, hq_o, gq_o = (nxt() for _ in range(4))
    if has_next:
        gs_o = nxt()

    gate = gate_r[...]
    fl = fl_r[...]
    hq = hq_r[...]
    if has_prev:
        pm = (gate == pgid_r[...]) & (fl == plvl_r[...])
        msgq = msgq_r[:, 0:H]
        hq = jnp.where(pm, _gru(msgq, hq, pwi, pwh, pbi, pbh), hq)
    nm = (gate == gid_r[...]) & (fl == lvl_r[...])
    msgs = msgs_r[:, 0:H]
    hs = hs_r[...]
    hs2 = jnp.where(nm, _gru(msgs, hs, swi, swh, sbi, sbh), hs)
    deg = msgs_r[:, H:H + 1]
    vf = jax.nn.relu(_dot(hs0_r[...], f1s[0]) + _dot(hf0_r[...], f1s[1]) + fb[0])
    vf = jax.nn.relu(_dot(vf, f23[0]) + fb[1])
    vf = _dot(vf, f23[1]) + fb[2]
    msgf = deg * vf
    hf = hf_r[...]
    hf2 = jnp.where(nm, _gru(msgf, hf, fwi, fwh, fbi, fbh), hf)
    u = _dot(hs2, q1s[0]) + _dot(hf2, q1s[1]) + _dot(hq, q1s[2]) + qb[0]
    u = jax.nn.relu(u)
    u = jax.nn.relu(_dot(u, q2[...]) + qb[1])
    gq_o[...] = _widen(_dot(u, q3[...]) + qb[2])
    if has_next:
        v = jax.nn.relu(_dot(hs2, n123[0]) + nb[0])
        v = jax.nn.relu(_dot(v, n123[1]) + nb[1])
        gs_o[...] = _widen(_dot(v, n123[2]) + nb[2])
    hs_o[...] = hs2
    hf_o[...] = hf2
    hq_o[...] = hq


def _epilogue_body(hf_r, hq_r, gate_r, fl_r, plvl_r, pgid_r,
                   msgq_r, pwi, pwh, pbi, pbh,
                   a01, a02, b01, b02, a11, a12, b11, b12,
                   at1, at2, bt1, bt2, a3s, b4, out_r):
    gate = gate_r[...]
    fl = fl_r[...]
    pm = (gate == pgid_r[...]) & (fl == plvl_r[...])
    msgq = msgq_r[:, 0:H]
    hq = hq_r[...]
    hq = jnp.where(pm, _gru(msgq, hq, pwi, pwh, pbi, pbh), hq)
    hf = hf_r[...]
    h0 = jax.nn.relu(_dot(hf, a01[...]) + b01[...])
    h0 = jax.nn.relu(_dot(h0, a02[...]) + b02[...])
    h1 = jax.nn.relu(_dot(hf, a11[...]) + b11[...])
    h1 = jax.nn.relu(_dot(h1, a12[...]) + b12[...])
    ht = jax.nn.relu(_dot(hq, at1[...]) + bt1[...])
    ht = jax.nn.relu(_dot(ht, at2[...]) + bt2[...])
    out_r[...] = (_dot(h0, a3s[0]) + _dot(h1, a3s[1]) + _dot(ht, a3s[2])
                  + b4[...])


def _tc_call(body, out_shapes, name):
    return pl.pallas_call(body, out_shape=out_shapes, name=name)


BN = 2000


def _row_spec(shape):
    if shape[0] in (N, NR):
        nd = len(shape)
        return pl.BlockSpec((BN,) + tuple(shape[1:]),
                            lambda i, _nd=nd: (i,) + (0,) * (_nd - 1))
    nd = len(shape)
    return pl.BlockSpec(tuple(shape), lambda i, _nd=nd: (0,) * _nd)


def _tc_call_grid(body, out_shapes, name, args):
    single = not isinstance(out_shapes, (list, tuple))
    outs = [out_shapes] if single else list(out_shapes)
    f = pl.pallas_call(
        body,
        out_shape=out_shapes,
        grid=(N // BN,),
        in_specs=[_row_spec(a.shape) for a in args],
        out_specs=(_row_spec(outs[0].shape) if single
                   else [_row_spec(o.shape) for o in outs]),
        name=name,
    )
    return f(*args)


def _gru_w(p):
    wi3 = jnp.stack(jnp.split(p['Wi'], 3, axis=1))
    wh3 = jnp.stack(jnp.split(p['Wh'], 3, axis=1))
    bi3 = p['bi'].reshape(3, 1, H)
    bh3 = p['bh'].reshape(3, 1, H)
    return [wi3, wh3, bi3, bh3]


def _mlp3_w(p):
    (w1, b1), (w2, b2), (w3, b3) = p
    return jnp.stack([w1, w2, w3]), jnp.stack(
        [b1.reshape(1, H), b2.reshape(1, H), b3.reshape(1, H)])


_NAMES = ['and', 'not', 'ff']


def kernel(x, edge_index, gate, forward_level, params):
    ei = edge_index.astype(jnp.int32).reshape(2 * E)
    g32 = gate.astype(jnp.int32)
    l32 = forward_level.astype(jnp.int32)
    es_src, es_dst, offs = _sort_edges(ei, g32, l32)
    gate2 = g32.reshape(N, 1)
    fl2 = l32.reshape(N, 1)
    zrows = jnp.zeros((NR, GW), jnp.float32)

    def sc11(v):
        return jnp.full((1, 1), v, jnp.int32)

    one = jnp.ones((1, 1), jnp.float32)
    s123_0, sb_0 = _mlp3_w(params['and_strc_aggr'])
    hs, hf, hq, gs, hs0r, hf0r = _tc_call(
        _prologue_body,
        [jax.ShapeDtypeStruct((N, H), jnp.float32)] * 3
        + [jax.ShapeDtypeStruct((N, GW), jnp.float32)]
        + [jax.ShapeDtypeStruct((1, H), jnp.float32)] * 2,
        "mlpgate_prologue")(
            one,
            params['hs_emd'][0][0], params['hs_emd'][0][1].reshape(1, H),
            params['hf_emd'][0][0], params['hf_emd'][0][1].reshape(1, H),
            params['hseq_emd'][0][0], params['hseq_emd'][0][1].reshape(1, H),
            s123_0, sb_0)

    msgs, _ = _scatter_pair(-1, gs, gs, es_src, es_dst, offs, zrows,
                            do_strc=True, do_seq=False)
    msgq = None
    prev = None
    nh = jax.ShapeDtypeStruct((N, H), jnp.float32)
    for t in range(12):
        lvl, gi = t // 3, (t % 3) + 1
        name = _NAMES[t % 3]
        args = [hs, hf, hq, gate2, fl2, sc11(lvl), sc11(gi), hs0r, hf0r,
                msgs]
        args += _gru_w(params[name + '_strc_gru'])
        args += _gru_w(params[name + '_func_gru'])
        (fw1, fb1), (fw2, fb2), (fw3, fb3) = params[name + '_func_aggr']
        args += [jnp.stack(jnp.split(fw1, 2, axis=0)),
                 jnp.stack([fw2, fw3]),
                 jnp.stack([fb1.reshape(1, H), fb2.reshape(1, H),
                            fb3.reshape(1, H)])]
        (qw1, qb1), (qw2, qb2), (qw3, qb3) = params[name + '_seq_aggr']
        args += [jnp.stack(jnp.split(qw1, 3, axis=0)), qw2, qw3,
                 jnp.stack([qb1.reshape(1, H), qb2.reshape(1, H),
                            qb3.reshape(1, H)])]
        has_prev = prev is not None
        if has_prev:
            plvl, pgi, pname = prev
            args += [sc11(plvl), sc11(pgi), msgq]
            args += _gru_w(params[pname + '_seq_gru'])
        has_next = t < 11
        if has_next:
            nname = _NAMES[(t + 1) % 3]
            n123, nb = _mlp3_w(params[nname + '_strc_aggr'])
            args += [n123, nb]
        ng = jax.ShapeDtypeStruct((N, GW), jnp.float32)
        outs = [nh, nh, nh, ng] + ([ng] if has_next else [])
        res = _tc_call_grid(
            functools.partial(_iter_body, has_prev, has_next), outs,
            f"mlpgate_iter_{int(has_prev)}{int(has_next)}", args)
        if has_next:
            hs, hf, hq, gq, gs = res
            msgs, msgq = _scatter_pair(t, gs, gq, es_src, es_dst, offs,
                                       zrows)
        else:
            hs, hf, hq, gq = res
            _, msgq = _scatter_pair(t, gq, gq, es_src, es_dst, offs, zrows,
                                    do_strc=False, do_seq=True)
        prev = (lvl, gi, name)

    (a01, b01), (a02, b02), (a03, b03) = params['readout_prob0']
    (a11, b11), (a12, b12), (a13, b13) = params['readout_prob1']
    (at1, bt1), (at2, bt2), (at3, bt3) = params['readout_trans']
    a3s = jnp.stack([
        jnp.pad(a03, ((0, 0), (0, 3))),
        jnp.pad(a13, ((0, 0), (1, 2))),
        jnp.pad(at3, ((0, 0), (2, 0))),
    ])
    b4 = jnp.concatenate([b03, b13, bt3]).reshape(1, 4)
    plvl, pgi, pname = prev
    eargs = [hf, hq, gate2, fl2, sc11(plvl), sc11(pgi),
             msgq, *_gru_w(params[pname + '_seq_gru']),
             a01, a02, b01.reshape(1, H), b02.reshape(1, H),
             a11, a12, b11.reshape(1, H), b12.reshape(1, H),
             at1, at2, bt1.reshape(1, H), bt2.reshape(1, H),
             a3s, b4]
    out = _tc_call_grid(_epilogue_body,
                        jax.ShapeDtypeStruct((N, 4), jnp.float32),
                        "mlpgate_epilogue", eargs)
    return out
